# baseline (speedup 1.0000x reference)
import functools

import jax
import jax.numpy as jnp
from jax import lax
from jax.experimental import pallas as pl
from jax.experimental.pallas import tpu as pltpu
from jax.experimental.pallas import tpu_sc as plsc

BATCH = 4096
SEQ = 200
EMBED_DIM = 64
PAD_DIM = 128
LANES = 16
NUM_WORKERS = 32
ROWS_PER_W = BATCH // NUM_WORKERS
CHUNK = 48
LAST_OFF = SEQ - CHUNK
NBUF = 2

VOCAB = 1000000
VBLK = 128
NBLK = (VOCAB + VBLK - 1) // VBLK
VOCAB_PAD = NBLK * VBLK
BLK_PER_W = NBLK // NUM_WORKERS
BLK_REM = NBLK - BLK_PER_W * NUM_WORKERS


def _fmt_body(tt_hbm, fmt_hbm, tin_v, tout_v, sem_in, sem_out):
    cid = lax.axis_index("c")
    sid = lax.axis_index("s")
    wid = sid * 2 + cid
    start = BLK_PER_W * wid + lax.min(wid, BLK_REM)
    count = BLK_PER_W + jnp.where(wid < BLK_REM, 1, 0)

    lane_iota = lax.iota(jnp.int32, LANES)
    sems = (sem_in, sem_out)

    def fire(j, buf):
        v0 = j * VBLK
        for i in range(EMBED_DIM // 8):
            pltpu.async_copy(
                tt_hbm.at[pl.ds(8 * i, 8), pl.ds(v0, VBLK)],
                tin_v.at[buf, pl.ds(8 * i, 8), :],
                sems[buf],
            )

    def drain_transpose_store(j, buf):
        v0 = j * VBLK
        for i in range(EMBED_DIM // 8):
            pltpu.make_async_copy(
                tt_hbm.at[pl.ds(8 * i, 8), pl.ds(v0, VBLK)],
                tin_v.at[buf, pl.ds(8 * i, 8), :],
                sems[buf],
            ).wait()
        for v in range(VBLK):
            vvec = jnp.full((LANES,), v, jnp.int32)
            for l in range(EMBED_DIM // LANES):
                dvec = lane_iota + l * LANES
                vals = plsc.load_gather(tin_v.at[buf], [dvec, vvec])
                plsc.store_scatter(tout_v.at[buf], [vvec, dvec], vals)
        pltpu.sync_copy(tout_v.at[buf],
                        fmt_hbm.at[pl.ds(v0, VBLK), :])

    fire(start, 0)

    def grp(g, _):
        k0 = 2 * g

        @pl.when(k0 + 1 < count)
        def _():
            fire(start + k0 + 1, 1)

        drain_transpose_store(start + k0, 0)

        @pl.when(k0 + 2 < count)
        def _():
            fire(start + k0 + 2, 0)

        @pl.when(k0 + 1 < count)
        def _():
            drain_transpose_store(start + k0 + 1, 1)

        return 0

    lax.fori_loop(0, (count + 1) // 2, grp, 0)


def _body(ids_hbm, lens_hbm, table_hbm, out_hbm, ids_v, lens_v, rows_v,
          out_v, sem0, sem1):
    cid = lax.axis_index("c")
    sid = lax.axis_index("s")
    wid = sid * 2 + cid
    base = wid * ROWS_PER_W
    sems = (sem0, sem1)

    pltpu.sync_copy(ids_hbm.at[pl.ds(base * SEQ, ROWS_PER_W * SEQ)], ids_v)
    pltpu.sync_copy(lens_hbm.at[pl.ds(base, ROWS_PER_W)],
                    lens_v.at[pl.ds(0, ROWS_PER_W)])

    lane_iota = lax.iota(jnp.int32, LANES)

    def nchunks(b):
        ln = lens_v[pl.ds(b, LANES)][0]
        return ln, lax.div(ln + (CHUNK - 1), CHUNK)

    def fire(b, buf):
        _, nch = nchunks(b)

        def chunk(c, _):
            off = lax.min(c * CHUNK, LAST_OFF)
            pltpu.async_copy(
                table_hbm.at[ids_v.at[pl.ds(b * SEQ + off, CHUNK)]],
                rows_v.at[buf, pl.ds(off, CHUNK), :],
                sems[buf],
            )
            return 0

        lax.fori_loop(0, nch, chunk, 0)

    def drain_sum(b, buf):
        ln, nch = nchunks(b)

        def dchunk(c, _):
            off = lax.min(c * CHUNK, LAST_OFF)
            pltpu.make_async_copy(
                table_hbm.at[ids_v.at[pl.ds(b * SEQ + off, CHUNK)]],
                rows_v.at[buf, pl.ds(off, CHUNK), :],
                sems[buf],
            ).wait()
            return 0

        lax.fori_loop(0, nch, dchunk, 0)

        def accum(s, acc):
            svec = jnp.full((LANES,), s, jnp.int32)
            return tuple(
                acc[l] + plsc.load_gather(
                    rows_v.at[buf], [svec, lane_iota + l * LANES])
                for l in range(4)
            )

        acc0 = tuple(jnp.zeros((LANES,), jnp.float32) for _ in range(4))
        acc = lax.fori_loop(0, ln, accum, acc0)

        den = jnp.full((LANES,), lax.max(ln, 1), jnp.int32).astype(jnp.float32)
        for l in range(4):
            out_v[pl.ds(b * EMBED_DIM + l * LANES, LANES)] = acc[l] / den

    for j in range(NBUF):
        fire(j, j)

    def group(i, _):
        b0 = NBUF * i
        for j in range(NBUF):
            b = b0 + j
            drain_sum(b, j)

            @pl.when(b + NBUF < ROWS_PER_W)
            def _():
                fire(b + NBUF, j)

        return 0

    lax.fori_loop(0, ROWS_PER_W // NBUF, group, 0)

    pltpu.sync_copy(out_v,
                    out_hbm.at[pl.ds(base * EMBED_DIM,
                                     ROWS_PER_W * EMBED_DIM)])


@jax.jit
def _pooled(token_ids, token_lens, table):
    ids_flat = token_ids.reshape(BATCH * SEQ)
    tt = table.T
    mesh = plsc.VectorSubcoreMesh(core_axis_name="c", subcore_axis_name="s")
    params = pltpu.CompilerParams(use_tc_tiling_on_sc=True,
                                  needs_layout_passes=False)

    fmt = functools.partial(
        pl.kernel,
        mesh=mesh,
        compiler_params=params,
        out_type=jax.ShapeDtypeStruct((VOCAB_PAD, PAD_DIM), jnp.float32),
        scratch_types=[
            pltpu.VMEM((2, EMBED_DIM, VBLK), jnp.float32),
            pltpu.VMEM((2, VBLK, PAD_DIM), jnp.float32),
            pltpu.SemaphoreType.DMA,
            pltpu.SemaphoreType.DMA,
        ],
    )(_fmt_body)
    table_fmt = fmt(tt)

    gather = functools.partial(
        pl.kernel,
        mesh=mesh,
        compiler_params=params,
        out_type=jax.ShapeDtypeStruct((BATCH * EMBED_DIM,), jnp.float32),
        scratch_types=[
            pltpu.VMEM((ROWS_PER_W * SEQ,), jnp.int32),
            pltpu.VMEM((ROWS_PER_W + LANES,), jnp.int32),
            pltpu.VMEM((NBUF, SEQ, PAD_DIM), jnp.float32),
            pltpu.VMEM((ROWS_PER_W * EMBED_DIM,), jnp.float32),
            pltpu.SemaphoreType.DMA,
            pltpu.SemaphoreType.DMA,
        ],
    )(_body)
    flat = gather(ids_flat, token_lens, table_fmt)
    return flat.reshape(BATCH, EMBED_DIM)


def kernel(token_ids, token_lens, table):
    return _pooled(token_ids, token_lens, table)


# speedup vs baseline: 1.4420x; 1.4420x over previous
import functools

import jax
import jax.numpy as jnp
from jax import lax
from jax.experimental import pallas as pl
from jax.experimental.pallas import tpu as pltpu
from jax.experimental.pallas import tpu_sc as plsc

BATCH = 4096
SEQ = 200
EMBED_DIM = 64
PAD_DIM = 128
LANES = 16
NUM_WORKERS = 32
ROWS_PER_W = BATCH // NUM_WORKERS
CHUNK = 48
LAST_OFF = SEQ - CHUNK
NBUF = 2

VOCAB = 1000000
VBLK = 128
NBLK = (VOCAB + VBLK - 1) // VBLK
VOCAB_PAD = NBLK * VBLK
BLK_PER_W = NBLK // NUM_WORKERS
BLK_REM = NBLK - BLK_PER_W * NUM_WORKERS


def _fmt_body(tt_hbm, fmt_hbm, tin_v, tout_v, sem_in, sem_out):
    cid = lax.axis_index("c")
    sid = lax.axis_index("s")
    wid = sid * 2 + cid
    start = BLK_PER_W * wid + lax.min(wid, BLK_REM)
    count = BLK_PER_W + jnp.where(wid < BLK_REM, 1, 0)

    lane_iota = lax.iota(jnp.int32, LANES)
    sems = (sem_in, sem_out)

    def fire(j, buf):
        v0 = j * VBLK
        for i in range(EMBED_DIM // 8):
            pltpu.async_copy(
                tt_hbm.at[pl.ds(8 * i, 8), pl.ds(v0, VBLK)],
                tin_v.at[buf, pl.ds(8 * i, 8), :],
                sems[buf],
            )

    def drain_transpose_store(j, buf):
        v0 = j * VBLK
        for i in range(EMBED_DIM // 8):
            pltpu.make_async_copy(
                tt_hbm.at[pl.ds(8 * i, 8), pl.ds(v0, VBLK)],
                tin_v.at[buf, pl.ds(8 * i, 8), :],
                sems[buf],
            ).wait()
        VB = 4
        for vq in range(0, VBLK, VB):
            vals = []
            for v in range(vq, vq + VB):
                vvec = jnp.full((LANES,), v, jnp.int32)
                for l in range(EMBED_DIM // LANES):
                    dvec = lane_iota + l * LANES
                    vals.append(plsc.load_gather(tin_v.at[buf],
                                                 [dvec, vvec]))
            k = 0
            for v in range(vq, vq + VB):
                for l in range(EMBED_DIM // LANES):
                    tout_v[buf, pl.ds(v * PAD_DIM + l * LANES, LANES)] = \
                        vals[k]
                    k += 1
        pltpu.sync_copy(tout_v.at[buf],
                        fmt_hbm.at[pl.ds(v0 * PAD_DIM, VBLK * PAD_DIM)])

    fire(start, 0)

    def grp(g, _):
        k0 = 2 * g

        @pl.when(k0 + 1 < count)
        def _():
            fire(start + k0 + 1, 1)

        drain_transpose_store(start + k0, 0)

        @pl.when(k0 + 2 < count)
        def _():
            fire(start + k0 + 2, 0)

        @pl.when(k0 + 1 < count)
        def _():
            drain_transpose_store(start + k0 + 1, 1)

        return 0

    lax.fori_loop(0, (count + 1) // 2, grp, 0)


def _body(ids_hbm, lens_hbm, table_hbm, out_hbm, ids_v, lens_v, rows_v,
          out_v, sem0, sem1):
    cid = lax.axis_index("c")
    sid = lax.axis_index("s")
    wid = sid * 2 + cid
    base = wid * ROWS_PER_W
    sems = (sem0, sem1)

    pltpu.sync_copy(ids_hbm.at[pl.ds(base * SEQ, ROWS_PER_W * SEQ)], ids_v)
    pltpu.sync_copy(lens_hbm.at[pl.ds(base, ROWS_PER_W)],
                    lens_v.at[pl.ds(0, ROWS_PER_W)])

    lane_iota = lax.iota(jnp.int32, LANES)

    def nchunks(b):
        ln = lens_v[pl.ds(b, LANES)][0]
        return ln, lax.div(ln + (CHUNK - 1), CHUNK)

    def fire(b, buf):
        _, nch = nchunks(b)

        def chunk(c, _):
            off = lax.min(c * CHUNK, LAST_OFF)
            pltpu.async_copy(
                table_hbm.at[ids_v.at[pl.ds(b * SEQ + off, CHUNK)]],
                rows_v.at[buf, pl.ds(off, CHUNK), :],
                sems[buf],
            )
            return 0

        lax.fori_loop(0, nch, chunk, 0)

    def drain_sum(b, buf):
        ln, nch = nchunks(b)

        def dchunk(c, _):
            off = lax.min(c * CHUNK, LAST_OFF)
            pltpu.make_async_copy(
                table_hbm.at[ids_v.at[pl.ds(b * SEQ + off, CHUNK)]],
                rows_v.at[buf, pl.ds(off, CHUNK), :],
                sems[buf],
            ).wait()
            return 0

        lax.fori_loop(0, nch, dchunk, 0)

        def accum(s, acc):
            svec = jnp.full((LANES,), s, jnp.int32)
            return tuple(
                acc[l] + plsc.load_gather(
                    rows_v.at[buf], [svec, lane_iota + l * LANES])
                for l in range(4)
            )

        acc0 = tuple(jnp.zeros((LANES,), jnp.float32) for _ in range(4))
        acc = lax.fori_loop(0, ln, accum, acc0)

        den = jnp.full((LANES,), lax.max(ln, 1), jnp.int32).astype(jnp.float32)
        for l in range(4):
            out_v[pl.ds(b * EMBED_DIM + l * LANES, LANES)] = acc[l] / den

    for j in range(NBUF):
        fire(j, j)

    def group(i, _):
        b0 = NBUF * i
        for j in range(NBUF):
            b = b0 + j
            drain_sum(b, j)

            @pl.when(b + NBUF < ROWS_PER_W)
            def _():
                fire(b + NBUF, j)

        return 0

    lax.fori_loop(0, ROWS_PER_W // NBUF, group, 0)

    pltpu.sync_copy(out_v,
                    out_hbm.at[pl.ds(base * EMBED_DIM,
                                     ROWS_PER_W * EMBED_DIM)])


@jax.jit
def _pooled(token_ids, token_lens, table):
    ids_flat = token_ids.reshape(BATCH * SEQ)
    tt = table.T
    mesh = plsc.VectorSubcoreMesh(core_axis_name="c", subcore_axis_name="s")
    params = pltpu.CompilerParams(use_tc_tiling_on_sc=True,
                                  needs_layout_passes=False)

    fmt = functools.partial(
        pl.kernel,
        mesh=mesh,
        compiler_params=params,
        out_type=jax.ShapeDtypeStruct((VOCAB_PAD * PAD_DIM,), jnp.float32),
        scratch_types=[
            pltpu.VMEM((2, EMBED_DIM, VBLK), jnp.float32),
            pltpu.VMEM((2, VBLK * PAD_DIM), jnp.float32),
            pltpu.SemaphoreType.DMA,
            pltpu.SemaphoreType.DMA,
        ],
    )(_fmt_body)
    table_fmt = fmt(tt).reshape(VOCAB_PAD, PAD_DIM)

    gather = functools.partial(
        pl.kernel,
        mesh=mesh,
        compiler_params=params,
        out_type=jax.ShapeDtypeStruct((BATCH * EMBED_DIM,), jnp.float32),
        scratch_types=[
            pltpu.VMEM((ROWS_PER_W * SEQ,), jnp.int32),
            pltpu.VMEM((ROWS_PER_W + LANES,), jnp.int32),
            pltpu.VMEM((NBUF, SEQ, PAD_DIM), jnp.float32),
            pltpu.VMEM((ROWS_PER_W * EMBED_DIM,), jnp.float32),
            pltpu.SemaphoreType.DMA,
            pltpu.SemaphoreType.DMA,
        ],
    )(_body)
    flat = gather(ids_flat, token_lens, table_fmt)
    return flat.reshape(BATCH, EMBED_DIM)


def kernel(token_ids, token_lens, table):
    return _pooled(token_ids, token_lens, table)


# speedup vs baseline: 3.0976x; 2.1482x over previous
import functools

import jax
import jax.numpy as jnp
from jax import lax
from jax.experimental import pallas as pl
from jax.experimental.pallas import tpu as pltpu
from jax.experimental.pallas import tpu_sc as plsc

BATCH = 4096
SEQ = 200
EMBED_DIM = 64
PAD_DIM = 128
LANES = 16
NUM_WORKERS = 32
ROWS_PER_W = BATCH // NUM_WORKERS
CHUNK = 48
LAST_OFF = SEQ - CHUNK
NBUF = 2


def _body(ids_hbm, lens_hbm, table_hbm, out_hbm, ids_v, lens_v, rows_v,
          out_v, sem0, sem1):
    cid = lax.axis_index("c")
    sid = lax.axis_index("s")
    wid = sid * 2 + cid
    base = wid * ROWS_PER_W
    sems = (sem0, sem1)

    pltpu.sync_copy(ids_hbm.at[pl.ds(base * SEQ, ROWS_PER_W * SEQ)], ids_v)
    pltpu.sync_copy(lens_hbm.at[pl.ds(base, ROWS_PER_W)],
                    lens_v.at[pl.ds(0, ROWS_PER_W)])

    lane_iota = lax.iota(jnp.int32, LANES)

    def nchunks(b):
        ln = lens_v[pl.ds(b, LANES)][0]
        return ln, lax.div(ln + (CHUNK - 1), CHUNK)

    def fire(b, buf):
        _, nch = nchunks(b)

        def chunk(c, _):
            off = lax.min(c * CHUNK, LAST_OFF)
            pltpu.async_copy(
                table_hbm.at[ids_v.at[pl.ds(b * SEQ + off, CHUNK)]],
                rows_v.at[buf, pl.ds(off, CHUNK), :],
                sems[buf],
            )
            return 0

        lax.fori_loop(0, nch, chunk, 0)

    def drain_sum(b, buf):
        ln, nch = nchunks(b)

        def dchunk(c, _):
            off = lax.min(c * CHUNK, LAST_OFF)
            pltpu.make_async_copy(
                table_hbm.at[ids_v.at[pl.ds(b * SEQ + off, CHUNK)]],
                rows_v.at[buf, pl.ds(off, CHUNK), :],
                sems[buf],
            ).wait()
            return 0

        lax.fori_loop(0, nch, dchunk, 0)

        def load4(s):
            svec = jnp.full((LANES,), s, jnp.int32)
            return [plsc.load_gather(rows_v.at[buf],
                                     [svec, lane_iota + l * LANES])
                    for l in range(4)]

        def accum2(i, acc):
            r0 = load4(2 * i)
            r1 = load4(2 * i + 1)
            return tuple(acc[l] + r0[l] + r1[l] for l in range(4))

        acc0 = tuple(jnp.zeros((LANES,), jnp.float32) for _ in range(4))
        acc = lax.fori_loop(0, lax.div(ln, 2), accum2, acc0)
        acc = lax.cond(
            lax.rem(ln, 2) == 1,
            lambda a: tuple(a[l] + v for l, v in enumerate(load4(ln - 1))),
            lambda a: a,
            acc,
        )

        den = jnp.full((LANES,), lax.max(ln, 1), jnp.int32).astype(jnp.float32)
        for l in range(4):
            out_v[pl.ds(b * EMBED_DIM + l * LANES, LANES)] = acc[l] / den

    for j in range(NBUF):
        fire(j, j)

    def group(i, _):
        b0 = NBUF * i
        for j in range(NBUF):
            b = b0 + j
            drain_sum(b, j)

            @pl.when(b + NBUF < ROWS_PER_W)
            def _():
                fire(b + NBUF, j)

        return 0

    lax.fori_loop(0, ROWS_PER_W // NBUF, group, 0)

    pltpu.sync_copy(out_v,
                    out_hbm.at[pl.ds(base * EMBED_DIM,
                                     ROWS_PER_W * EMBED_DIM)])


@jax.jit
def _pooled(token_ids, token_lens, table):
    ids_flat = token_ids.reshape(BATCH * SEQ)
    table_pad = jnp.pad(table, ((0, 0), (0, PAD_DIM - EMBED_DIM)))
    mesh = plsc.VectorSubcoreMesh(core_axis_name="c", subcore_axis_name="s")
    f = functools.partial(
        pl.kernel,
        mesh=mesh,
        compiler_params=pltpu.CompilerParams(use_tc_tiling_on_sc=True,
                                             needs_layout_passes=False),
        out_type=jax.ShapeDtypeStruct((BATCH * EMBED_DIM,), jnp.float32),
        scratch_types=[
            pltpu.VMEM((ROWS_PER_W * SEQ,), jnp.int32),
            pltpu.VMEM((ROWS_PER_W + LANES,), jnp.int32),
            pltpu.VMEM((NBUF, SEQ, PAD_DIM), jnp.float32),
            pltpu.VMEM((ROWS_PER_W * EMBED_DIM,), jnp.float32),
            pltpu.SemaphoreType.DMA,
            pltpu.SemaphoreType.DMA,
        ],
    )(_body)
    flat = f(ids_flat, token_lens, table_pad)
    return flat.reshape(BATCH, EMBED_DIM)


def kernel(token_ids, token_lens, table):
    return _pooled(token_ids, token_lens, table)
